# Initial kernel scaffold; baseline (speedup 1.0000x reference)
import functools

import jax
import jax.numpy as jnp
from jax import lax
from jax.experimental import pallas as pl
from jax.experimental.pallas import tpu as pltpu
from jax.experimental.pallas import tpu_sc as plsc

N_NODES = 10000
N_EDGES = 320000
D = 128

NC = 2
NS = 16
NW = NC * NS
LANES = 16

EW = N_EDGES // NW
N_PAD = 10240
SL = N_PAD // NW
CH = 64
NCH = SL // CH
ZROW = N_NODES

_mesh = plsc.VectorSubcoreMesh(core_axis_name="c", subcore_axis_name="s")


@functools.partial(
    pl.kernel,
    mesh=_mesh,
    out_type=jax.ShapeDtypeStruct((NW, N_PAD), jnp.int32),
    scratch_types=[
        pltpu.VMEM((EW,), jnp.int32),
        pltpu.VMEM((N_PAD,), jnp.int32),
    ],
)
def _lastedge_kernel(dst_hbm, local_all, dst_v, last_v):
    wid = lax.axis_index("s") * NC + lax.axis_index("c")
    neg1 = jnp.full((LANES,), -1, jnp.int32)

    def init_body(j, carry):
        last_v[pl.ds(j * LANES, LANES)] = neg1
        return carry

    lax.fori_loop(0, N_PAD // LANES, init_body, 0)

    pltpu.sync_copy(dst_hbm.at[pl.ds(wid * EW, EW)], dst_v)
    ebase = wid * EW
    iota = lax.iota(jnp.int32, LANES)

    def chunk_body(c, carry):
        d16 = dst_v[pl.ds(c * LANES, LANES)]
        e16 = ebase + c * LANES + iota
        plsc.store_scatter(last_v, [d16], e16)
        got0 = plsc.load_gather(last_v, [d16])

        def fix_cond(got):
            return jnp.any(e16 > got)

        def fix_body(got):
            plsc.store_scatter(last_v, [d16], e16, mask=e16 > got)
            return plsc.load_gather(last_v, [d16])

        lax.while_loop(fix_cond, fix_body, got0)
        return carry

    lax.fori_loop(0, EW // LANES, chunk_body, 0)
    pltpu.sync_copy(last_v, local_all.at[wid])


@functools.partial(
    pl.kernel,
    mesh=_mesh,
    out_type=jax.ShapeDtypeStruct((N_PAD, D), jnp.float32),
    scratch_types=[
        pltpu.VMEM((NW, SL), jnp.int32),
        pltpu.VMEM((NCH, CH), jnp.int32),
        pltpu.VMEM((NCH, CH), jnp.int32),
        pltpu.VMEM((NCH, CH), jnp.int32),
        pltpu.VMEM((CH, D), jnp.float32),
        pltpu.SemaphoreType.DMA,
    ],
)
def _agg_kernel(local_all, src_hbm, xpad_hbm, agg_hbm,
                sl_v, eidx_v, srcv_v, safe_v, rows_v, sem):
    wid = lax.axis_index("s") * NC + lax.axis_index("c")
    base = wid * SL

    pltpu.sync_copy(local_all.at[:, pl.ds(base, SL)], sl_v)

    for j in range(SL // LANES):
        acc = sl_v[0, pl.ds(j * LANES, LANES)]
        for w in range(1, NW):
            acc = jnp.maximum(acc, sl_v[w, pl.ds(j * LANES, LANES)])
        k, o = divmod(j * LANES, CH)
        eidx_v[k, pl.ds(o, LANES)] = jnp.maximum(acc, 0)
        safe_v[k, pl.ds(o, LANES)] = acc

    for k in range(NCH):
        pltpu.async_copy(src_hbm.at[eidx_v.at[k]], srcv_v.at[k], sem).wait()

    for k in range(NCH):
        for o in range(0, CH, LANES):
            lastv = safe_v[k, pl.ds(o, LANES)]
            s = srcv_v[k, pl.ds(o, LANES)]
            safe_v[k, pl.ds(o, LANES)] = jnp.where(lastv >= 0, s, ZROW)

    for k in range(NCH):
        pltpu.async_copy(xpad_hbm.at[safe_v.at[k]], rows_v, sem).wait()
        pltpu.sync_copy(rows_v, agg_hbm.at[pl.ds(base + k * CH, CH)])


def _mm_body(x_ref, agg_ref, w_ref, b_ref, o_ref):
    w1 = w_ref[:, :D]
    w2 = w_ref[:, D:]
    dn = (((1,), (1,)), ((), ()))
    o_ref[...] = (
        lax.dot_general(x_ref[...], w1, dn, preferred_element_type=jnp.float32)
        + lax.dot_general(agg_ref[...], w2, dn, preferred_element_type=jnp.float32)
        + b_ref[...]
    )


_ROWS_BLK = 1000


def _matmul(x, agg, W, b2d):
    grid = (N_NODES // _ROWS_BLK,)
    return pl.pallas_call(
        _mm_body,
        grid=grid,
        in_specs=[
            pl.BlockSpec((_ROWS_BLK, D), lambda i: (i, 0)),
            pl.BlockSpec((_ROWS_BLK, D), lambda i: (i, 0)),
            pl.BlockSpec((D, 2 * D), lambda i: (0, 0)),
            pl.BlockSpec((1, D), lambda i: (0, 0)),
        ],
        out_specs=pl.BlockSpec((_ROWS_BLK, D), lambda i: (i, 0)),
        out_shape=jax.ShapeDtypeStruct((N_NODES, D), jnp.float32),
    )(x, agg, W, b2d)


@jax.jit
def kernel(x, edge_index, W, b):
    dst = edge_index[0]
    src = edge_index[1]
    xpad = jnp.concatenate([x, jnp.zeros((LANES, D), x.dtype)], axis=0)
    local_all = _lastedge_kernel(dst)
    agg = _agg_kernel(local_all, src, xpad)
    return _matmul(x, agg[:N_NODES], W, b.reshape(1, D))


# speedup vs baseline: 17.8817x; 17.8817x over previous
import functools

import jax
import jax.numpy as jnp
from jax import lax
from jax.experimental import pallas as pl
from jax.experimental.pallas import tpu as pltpu
from jax.experimental.pallas import tpu_sc as plsc

N_NODES = 10000
N_EDGES = 320000
D = 128

NC = 2
NS = 16
NW = NC * NS
LANES = 16

EW = N_EDGES // NW
N_PAD = 10240
SL = N_PAD // NW
CH = 64
NCH = SL // CH
ZROW = N_NODES

_mesh = plsc.VectorSubcoreMesh(core_axis_name="c", subcore_axis_name="s")
_sc_params = pltpu.CompilerParams(
    needs_layout_passes=False, use_tc_tiling_on_sc=False
)


@functools.partial(
    pl.kernel,
    mesh=_mesh,
    out_type=jax.ShapeDtypeStruct((NW, N_PAD), jnp.int32),
    compiler_params=_sc_params,
    scratch_types=[
        pltpu.VMEM((EW,), jnp.int32),
        pltpu.VMEM((N_PAD,), jnp.int32),
    ],
)
def _lastedge_kernel(dst_hbm, local_all, dst_v, last_v):
    wid = lax.axis_index("s") * NC + lax.axis_index("c")
    neg1 = jnp.full((LANES,), -1, jnp.int32)

    def init_body(j, carry):
        last_v[pl.ds(j * LANES, LANES)] = neg1
        return carry

    lax.fori_loop(0, N_PAD // LANES, init_body, 0)

    pltpu.sync_copy(dst_hbm.at[pl.ds(wid * EW, EW)], dst_v)
    ebase = wid * EW
    iota = lax.iota(jnp.int32, LANES)

    def chunk_body(c, carry):
        d16 = dst_v[pl.ds(c * LANES, LANES)]
        e16 = ebase + c * LANES + iota
        plsc.store_scatter(last_v, [d16], e16)
        got0 = plsc.load_gather(last_v, [d16])

        def fix_cond(got):
            return jnp.any(e16 > got)

        def fix_body(got):
            plsc.store_scatter(last_v, [d16], e16, mask=e16 > got)
            return plsc.load_gather(last_v, [d16])

        lax.while_loop(fix_cond, fix_body, got0)
        return carry

    lax.fori_loop(0, EW // LANES, chunk_body, 0)
    pltpu.sync_copy(last_v, local_all.at[wid])


@functools.partial(
    pl.kernel,
    mesh=_mesh,
    out_type=jax.ShapeDtypeStruct((N_PAD, D), jnp.float32),
    compiler_params=_sc_params,
    scratch_types=[
        pltpu.VMEM((NW, SL), jnp.int32),
        pltpu.VMEM((NCH, CH), jnp.int32),
        pltpu.VMEM((NCH, CH), jnp.int32),
        pltpu.VMEM((NCH, CH), jnp.int32),
        pltpu.VMEM((CH, D), jnp.float32),
        pltpu.SemaphoreType.DMA,
    ],
)
def _agg_kernel(local_all, src_hbm, xpad_hbm, agg_hbm,
                sl_v, eidx_v, srcv_v, safe_v, rows_v, sem):
    wid = lax.axis_index("s") * NC + lax.axis_index("c")
    base = wid * SL

    pltpu.sync_copy(local_all.at[:, pl.ds(base, SL)], sl_v)

    for j in range(SL // LANES):
        acc = sl_v[0, pl.ds(j * LANES, LANES)]
        for w in range(1, NW):
            acc = jnp.maximum(acc, sl_v[w, pl.ds(j * LANES, LANES)])
        k, o = divmod(j * LANES, CH)
        eidx_v[k, pl.ds(o, LANES)] = jnp.maximum(acc, 0)
        safe_v[k, pl.ds(o, LANES)] = acc

    for k in range(NCH):
        pltpu.async_copy(src_hbm.at[eidx_v.at[k]], srcv_v.at[k], sem).wait()

    for k in range(NCH):
        for o in range(0, CH, LANES):
            lastv = safe_v[k, pl.ds(o, LANES)]
            s = srcv_v[k, pl.ds(o, LANES)]
            safe_v[k, pl.ds(o, LANES)] = jnp.where(lastv >= 0, s, ZROW)

    for k in range(NCH):
        pltpu.async_copy(xpad_hbm.at[safe_v.at[k]], rows_v, sem).wait()
        pltpu.sync_copy(rows_v, agg_hbm.at[pl.ds(base + k * CH, CH)])


def _mm_body(x_ref, agg_ref, w_ref, b_ref, o_ref):
    w1 = w_ref[:, :D]
    w2 = w_ref[:, D:]
    dn = (((1,), (1,)), ((), ()))
    o_ref[...] = (
        lax.dot_general(x_ref[...], w1, dn, preferred_element_type=jnp.float32)
        + lax.dot_general(agg_ref[...], w2, dn, preferred_element_type=jnp.float32)
        + b_ref[...]
    )


_ROWS_BLK = 1000


def _matmul(x, agg, W, b2d):
    grid = (N_NODES // _ROWS_BLK,)
    return pl.pallas_call(
        _mm_body,
        grid=grid,
        in_specs=[
            pl.BlockSpec((_ROWS_BLK, D), lambda i: (i, 0)),
            pl.BlockSpec((_ROWS_BLK, D), lambda i: (i, 0)),
            pl.BlockSpec((D, 2 * D), lambda i: (0, 0)),
            pl.BlockSpec((1, D), lambda i: (0, 0)),
        ],
        out_specs=pl.BlockSpec((_ROWS_BLK, D), lambda i: (i, 0)),
        out_shape=jax.ShapeDtypeStruct((N_NODES, D), jnp.float32),
    )(x, agg, W, b2d)


@jax.jit
def kernel(x, edge_index, W, b):
    dst = edge_index[0]
    src = edge_index[1]
    xpad = jnp.concatenate([x, jnp.zeros((LANES, D), x.dtype)], axis=0)
    local_all = _lastedge_kernel(dst)
    agg = _agg_kernel(local_all, src, xpad)
    return _matmul(x, agg[:N_NODES], W, b.reshape(1, D))


# speedup vs baseline: 20.4988x; 1.1464x over previous
import functools

import jax
import jax.numpy as jnp
from jax import lax
from jax.experimental import pallas as pl
from jax.experimental.pallas import tpu as pltpu
from jax.experimental.pallas import tpu_sc as plsc

N_NODES = 10000
N_EDGES = 320000
D = 128

NC = 2
NS = 16
NW = NC * NS
LANES = 16

NBLK = N_EDGES // D
BPW = 79
N_PAD = 10240
SL = N_PAD // NW
CH = 64
NCH = SL // CH
ZROW = N_NODES

_mesh = plsc.VectorSubcoreMesh(core_axis_name="c", subcore_axis_name="s")
_sc_params = pltpu.CompilerParams(
    needs_layout_passes=False, use_tc_tiling_on_sc=False
)


@functools.partial(
    pl.kernel,
    mesh=_mesh,
    out_type=(
        jax.ShapeDtypeStruct((NW, N_PAD), jnp.int32),
        jax.ShapeDtypeStruct((NW, N_PAD), jnp.int32),
    ),
    compiler_params=_sc_params,
    scratch_types=[
        pltpu.VMEM((BPW, 2, D), jnp.int32),
        pltpu.VMEM((N_PAD,), jnp.int32),
        pltpu.VMEM((N_PAD,), jnp.int32),
    ],
)
def _lastedge_kernel(ei_hbm, last_all, src_all, ei_v, last_v, src_v):
    wid = lax.axis_index("s") * NC + lax.axis_index("c")
    start = jnp.minimum(wid * (NBLK // NW) + jnp.minimum(wid, NBLK % NW),
                        NBLK - BPW)
    neg1 = jnp.full((LANES,), -1, jnp.int32)

    def init_body(j, carry):
        last_v[pl.ds(j * LANES, LANES)] = neg1
        return carry

    lax.fori_loop(0, N_PAD // LANES, init_body, 0)

    pltpu.sync_copy(ei_hbm.at[pl.ds(start, BPW)], ei_v)
    iota = lax.iota(jnp.int32, LANES)

    def blk_body(j, carry):
        ebase = (start + j) * D

        def sub_body(o, carry2):
            d16 = ei_v[j, 0, pl.ds(o * LANES, LANES)]
            e16 = ebase + o * LANES + iota
            plsc.store_scatter(last_v, [d16], e16)
            got0 = plsc.load_gather(last_v, [d16])

            def fix_cond(got):
                return jnp.any(e16 > got)

            def fix_body(got):
                plsc.store_scatter(last_v, [d16], e16, mask=e16 > got)
                return plsc.load_gather(last_v, [d16])

            got = lax.while_loop(fix_cond, fix_body, got0)
            s16 = ei_v[j, 1, pl.ds(o * LANES, LANES)]
            plsc.store_scatter(src_v, [d16], s16, mask=e16 == got)
            return carry2

        lax.fori_loop(0, D // LANES, sub_body, 0)
        return carry

    lax.fori_loop(0, BPW, blk_body, 0)
    pltpu.sync_copy(last_v, last_all.at[wid])
    pltpu.sync_copy(src_v, src_all.at[wid])


@functools.partial(
    pl.kernel,
    mesh=_mesh,
    out_type=jax.ShapeDtypeStruct((N_PAD, D), jnp.float32),
    compiler_params=_sc_params,
    scratch_types=[
        pltpu.VMEM((NW, SL), jnp.int32),
        pltpu.VMEM((NW, SL), jnp.int32),
        pltpu.VMEM((NCH, CH), jnp.int32),
        pltpu.VMEM((SL, D), jnp.float32),
        pltpu.SemaphoreType.DMA,
    ],
)
def _agg_kernel(last_all, src_all, xpad_hbm, agg_hbm,
                lsl_v, ssl_v, safe_v, rows_v, sem):
    wid = lax.axis_index("s") * NC + lax.axis_index("c")
    base = wid * SL

    cp1 = pltpu.async_copy(last_all.at[:, pl.ds(base, SL)], lsl_v, sem)
    cp2 = pltpu.async_copy(src_all.at[:, pl.ds(base, SL)], ssl_v, sem)
    cp1.wait()
    cp2.wait()

    for j in range(SL // LANES):
        sl = pl.ds(j * LANES, LANES)
        best = lsl_v[0, sl]
        bsrc = ssl_v[0, sl]
        for w in range(1, NW):
            lw = lsl_v[w, sl]
            sw = ssl_v[w, sl]
            m = lw > best
            best = jnp.maximum(best, lw)
            bsrc = jnp.where(m, sw, bsrc)
        k, o = divmod(j * LANES, CH)
        safe_v[k, pl.ds(o, LANES)] = jnp.where(best >= 0, bsrc, ZROW)

    copies = [
        pltpu.async_copy(
            xpad_hbm.at[safe_v.at[k]], rows_v.at[pl.ds(k * CH, CH)], sem
        )
        for k in range(NCH)
    ]
    for cp in copies:
        cp.wait()
    pltpu.sync_copy(rows_v, agg_hbm.at[pl.ds(base, SL)])


def _mm_body(x_ref, agg_ref, w_ref, b_ref, o_ref):
    w1 = w_ref[:, :D]
    w2 = w_ref[:, D:]
    dn = (((1,), (1,)), ((), ()))
    o_ref[...] = (
        lax.dot_general(x_ref[...], w1, dn, preferred_element_type=jnp.float32)
        + lax.dot_general(agg_ref[...], w2, dn, preferred_element_type=jnp.float32)
        + b_ref[...]
    )


_ROWS_BLK = 1000


def _matmul(x, agg, W, b2d):
    grid = (N_NODES // _ROWS_BLK,)
    return pl.pallas_call(
        _mm_body,
        grid=grid,
        in_specs=[
            pl.BlockSpec((_ROWS_BLK, D), lambda i: (i, 0)),
            pl.BlockSpec((_ROWS_BLK, D), lambda i: (i, 0)),
            pl.BlockSpec((D, 2 * D), lambda i: (0, 0)),
            pl.BlockSpec((1, D), lambda i: (0, 0)),
        ],
        out_specs=pl.BlockSpec((_ROWS_BLK, D), lambda i: (i, 0)),
        out_shape=jax.ShapeDtypeStruct((N_NODES, D), jnp.float32),
    )(x, agg, W, b2d)


@jax.jit
def kernel(x, edge_index, W, b):
    ei_t = jnp.transpose(edge_index.reshape(2, NBLK, D), (1, 0, 2))
    xpad = jnp.concatenate([x, jnp.zeros((LANES, D), x.dtype)], axis=0)
    last_all, src_all = _lastedge_kernel(ei_t)
    agg = _agg_kernel(last_all, src_all, xpad)
    return _matmul(x, agg, W, b.reshape(1, D))


# speedup vs baseline: 30.4930x; 1.4876x over previous
import functools

import jax
import jax.numpy as jnp
from jax import lax
from jax.experimental import pallas as pl
from jax.experimental.pallas import tpu as pltpu
from jax.experimental.pallas import tpu_sc as plsc

N_NODES = 10000
N_EDGES = 320000
D = 128

NC = 2
NS = 16
NW = NC * NS
LANES = 16

NBLK = N_EDGES // D
BPW = 79
N_PAD = 10240
SL = N_PAD // NW
CH = 64
NCH = SL // CH
ZROW = N_NODES

_mesh = plsc.VectorSubcoreMesh(core_axis_name="c", subcore_axis_name="s")
_sc_params = pltpu.CompilerParams(
    needs_layout_passes=False, use_tc_tiling_on_sc=False
)


@functools.partial(
    pl.kernel,
    mesh=_mesh,
    out_type=jax.ShapeDtypeStruct((NW, N_PAD), jnp.int32),
    compiler_params=_sc_params,
    scratch_types=[
        pltpu.VMEM((BPW, 2, D), jnp.int32),
        pltpu.VMEM((N_PAD,), jnp.int32),
    ],
)
def _lastsrc_kernel(ei_hbm, src_all, ei_v, src_v):
    wid = lax.axis_index("s") * NC + lax.axis_index("c")
    start = jnp.minimum(wid * (NBLK // NW) + jnp.minimum(wid, NBLK % NW),
                        NBLK - BPW)
    neg1 = jnp.full((LANES,), -1, jnp.int32)

    def init_body(j, carry):
        src_v[pl.ds(j * LANES, LANES)] = neg1
        return carry

    lax.fori_loop(0, N_PAD // LANES, init_body, 0)

    pltpu.sync_copy(ei_hbm.at[pl.ds(start, BPW)], ei_v)

    def blk_body(j, carry):
        def sub_body(o, carry2):
            d16 = ei_v[j, 0, pl.ds(o * LANES, LANES)]
            s16 = ei_v[j, 1, pl.ds(o * LANES, LANES)]
            plsc.store_scatter(src_v, [d16], s16)
            return carry2

        lax.fori_loop(0, D // LANES, sub_body, 0)
        return carry

    lax.fori_loop(0, BPW, blk_body, 0)
    pltpu.sync_copy(src_v, src_all.at[wid])


@functools.partial(
    pl.kernel,
    mesh=_mesh,
    out_type=jax.ShapeDtypeStruct((N_PAD, D), jnp.float32),
    compiler_params=_sc_params,
    scratch_types=[
        pltpu.VMEM((NW, SL), jnp.int32),
        pltpu.VMEM((SL,), jnp.int32),
        pltpu.VMEM((SL, D), jnp.float32),
        pltpu.SemaphoreType.DMA,
    ],
)
def _agg_kernel(src_all, xpad_hbm, agg_hbm, ssl_v, safe_v, rows_v, sem):
    wid = lax.axis_index("s") * NC + lax.axis_index("c")
    base = wid * SL

    pltpu.sync_copy(src_all.at[:, pl.ds(base, SL)], ssl_v)

    zrow = jnp.full((LANES,), ZROW, jnp.int32)

    def comb_body(j, carry):
        sl = pl.ds(j * LANES, LANES)
        bsrc = zrow
        for w in range(NW):
            sw = ssl_v[w, sl]
            bsrc = jnp.where(sw >= 0, sw, bsrc)
        safe_v[sl] = bsrc
        return carry

    lax.fori_loop(0, SL // LANES, comb_body, 0)

    copies = [
        pltpu.async_copy(
            xpad_hbm.at[safe_v.at[pl.ds(k * CH, CH)]],
            rows_v.at[pl.ds(k * CH, CH)],
            sem,
        )
        for k in range(NCH)
    ]
    for cp in copies:
        cp.wait()
    pltpu.sync_copy(rows_v, agg_hbm.at[pl.ds(base, SL)])


def _mm1_body(x_ref, w_ref, b_ref, o_ref):
    dn = (((1,), (1,)), ((), ()))
    o_ref[...] = (
        lax.dot_general(x_ref[...], w_ref[:, :D], dn,
                        preferred_element_type=jnp.float32)
        + b_ref[...]
    )


def _mm2_body(y1_ref, agg_ref, w_ref, o_ref):
    dn = (((1,), (1,)), ((), ()))
    o_ref[...] = y1_ref[...] + lax.dot_general(
        agg_ref[...], w_ref[:, D:], dn, preferred_element_type=jnp.float32
    )


_ROWS_BLK = 2000
_GRID = (N_NODES // _ROWS_BLK,)
_ROW_SPEC = pl.BlockSpec((_ROWS_BLK, D), lambda i: (i, 0))
_W_SPEC = pl.BlockSpec((D, 2 * D), lambda i: (0, 0))
_OUT_TYPE = jax.ShapeDtypeStruct((N_NODES, D), jnp.float32)


def _mm1(x, W, b2d):
    return pl.pallas_call(
        _mm1_body,
        grid=_GRID,
        in_specs=[_ROW_SPEC, _W_SPEC, pl.BlockSpec((1, D), lambda i: (0, 0))],
        out_specs=_ROW_SPEC,
        out_shape=_OUT_TYPE,
    )(x, W, b2d)


def _mm2(y1, agg, W):
    return pl.pallas_call(
        _mm2_body,
        grid=_GRID,
        in_specs=[_ROW_SPEC, _ROW_SPEC, _W_SPEC],
        out_specs=_ROW_SPEC,
        out_shape=_OUT_TYPE,
    )(y1, agg, W)


@jax.jit
def kernel(x, edge_index, W, b):
    ei_t = jnp.transpose(edge_index.reshape(2, NBLK, D), (1, 0, 2))
    xpad = jnp.concatenate([x, jnp.zeros((LANES, D), x.dtype)], axis=0)
    src_all = _lastsrc_kernel(ei_t)
    agg = _agg_kernel(src_all, xpad)
    y1 = _mm1(x, W, b.reshape(1, D))
    return _mm2(y1, agg, W)
